# Initial kernel scaffold; baseline (speedup 1.0000x reference)
import jax
import jax.numpy as jnp
from jax.experimental import pallas as pl


def kernel(g, x, wg, gamma_g, beta_g, wx, gamma_x, beta_x, wpsi, gamma_p, beta_p):
    raise NotImplementedError("write your pallas kernel here")


# speedup vs baseline: 1.1279x; 1.1279x over previous
import jax
import jax.numpy as jnp
from jax.experimental import pallas as pl
from jax.experimental.pallas import tpu as pltpu

_BN_EPS = 1e-5


def _proj_stats_kernel(g_ref, x_ref, wg_ref, wx_ref, y_ref, st_ref):
    f_int = wg_ref.shape[0]
    gb = g_ref[0].astype(jnp.bfloat16)
    xb = x_ref[0].astype(jnp.bfloat16)
    u = jnp.dot(wg_ref[...].astype(jnp.bfloat16), gb,
                preferred_element_type=jnp.float32)
    v = jnp.dot(wx_ref[...].astype(jnp.bfloat16), xb,
                preferred_element_type=jnp.float32)
    y_ref[0, :f_int] = u.astype(jnp.bfloat16)
    y_ref[0, f_int:] = v.astype(jnp.bfloat16)
    st_ref[0] = jnp.concatenate(
        [jnp.sum(u, axis=1, keepdims=True),
         jnp.sum(u * u, axis=1, keepdims=True),
         jnp.sum(v, axis=1, keepdims=True),
         jnp.sum(v * v, axis=1, keepdims=True)], axis=1)


def _psi_kernel(y_ref, st_ref, bn1_ref, wpsi_ref, inv_ref, psi_ref, ps_ref):
    f_int = bn1_ref.shape[0]
    inv = inv_ref[0, 0]
    s = jnp.sum(st_ref[...], axis=0)
    mu = s[:, 0:1] * inv
    vu = s[:, 1:2] * inv - mu * mu
    su = bn1_ref[:, 0:1] * jax.lax.rsqrt(vu + _BN_EPS)
    hu = bn1_ref[:, 1:2] - mu * su
    mv = s[:, 2:3] * inv
    vv = s[:, 3:4] * inv - mv * mv
    sv = bn1_ref[:, 2:3] * jax.lax.rsqrt(vv + _BN_EPS)
    hv = bn1_ref[:, 3:4] - mv * sv
    u = y_ref[0, :f_int]
    v = y_ref[0, f_int:]
    a = jnp.maximum(u * su + v * sv + (hu + hv), 0.0)
    p = jnp.dot(wpsi_ref[...], a, preferred_element_type=jnp.float32)
    psi_ref[0] = p
    ps_ref[0] = jnp.concatenate(
        [jnp.sum(p, axis=1, keepdims=True),
         jnp.sum(p * p, axis=1, keepdims=True)], axis=1)


def _gate_kernel(x_ref, psi_ref, ps_ref, bnp_ref, inv_ref, o_ref):
    inv = inv_ref[0, 0]
    s = jnp.sum(ps_ref[...], axis=0)
    m = s[:, 0:1] * inv
    var = s[:, 1:2] * inv - m * m
    sc = bnp_ref[:, 0:1] * jax.lax.rsqrt(var + _BN_EPS)
    sh = bnp_ref[:, 1:2] - m * sc
    z = psi_ref[0] * sc + sh
    gate = 1.0 / (1.0 + jnp.exp(-z))
    o_ref[0] = x_ref[0] * gate


def kernel(g, x, wg, gamma_g, beta_g, wx, gamma_x, beta_x, wpsi,
           gamma_p, beta_p):
    N, F_l, H, W = g.shape
    _, F_g, _, _ = x.shape
    F_int = wg.shape[0]
    M = H * W

    g3 = g.reshape(N, F_l, M)
    x3 = x.reshape(N, F_g, M)
    bn1 = jnp.stack([gamma_g, beta_g, gamma_x, beta_x], axis=1)
    bnp = jnp.stack([gamma_p, beta_p], axis=1)
    inv = jnp.full((1, 1), 1.0 / (N * M), jnp.float32)

    def vconst(shape):
        return pl.BlockSpec(shape, lambda n: (0,) * len(shape))

    y, st = pl.pallas_call(
        _proj_stats_kernel,
        out_shape=(jax.ShapeDtypeStruct((N, 2 * F_int, M), jnp.bfloat16),
                   jax.ShapeDtypeStruct((N, F_int, 4), jnp.float32)),
        grid=(N,),
        in_specs=[
            pl.BlockSpec((1, F_l, M), lambda n: (n, 0, 0)),
            pl.BlockSpec((1, F_g, M), lambda n: (n, 0, 0)),
            vconst((F_int, F_l)),
            vconst((F_int, F_g)),
        ],
        out_specs=(pl.BlockSpec((1, 2 * F_int, M), lambda n: (n, 0, 0)),
                   pl.BlockSpec((1, F_int, 4), lambda n: (n, 0, 0))),
        compiler_params=pltpu.CompilerParams(
            dimension_semantics=("parallel",)),
    )(g3, x3, wg, wx)

    psi, ps = pl.pallas_call(
        _psi_kernel,
        out_shape=(jax.ShapeDtypeStruct((N, 1, M), jnp.float32),
                   jax.ShapeDtypeStruct((N, 1, 2), jnp.float32)),
        grid=(N,),
        in_specs=[
            pl.BlockSpec((1, 2 * F_int, M), lambda n: (n, 0, 0)),
            vconst((N, F_int, 4)),
            vconst((F_int, 4)),
            vconst((1, F_int)),
            vconst((1, 1)),
        ],
        out_specs=(pl.BlockSpec((1, 1, M), lambda n: (n, 0, 0)),
                   pl.BlockSpec((1, 1, 2), lambda n: (n, 0, 0))),
        compiler_params=pltpu.CompilerParams(
            dimension_semantics=("parallel",)),
    )(y, st, bn1, wpsi, inv)

    out = pl.pallas_call(
        _gate_kernel,
        out_shape=jax.ShapeDtypeStruct((N, F_g, M), jnp.float32),
        grid=(N,),
        in_specs=[
            pl.BlockSpec((1, F_g, M), lambda n: (n, 0, 0)),
            pl.BlockSpec((1, 1, M), lambda n: (n, 0, 0)),
            vconst((N, 1, 2)),
            vconst((1, 2)),
            vconst((1, 1)),
        ],
        out_specs=pl.BlockSpec((1, F_g, M), lambda n: (n, 0, 0)),
        compiler_params=pltpu.CompilerParams(
            dimension_semantics=("parallel",)),
    )(x3, psi, ps, bnp, inv)

    return out.reshape(N, F_g, H, W)
